# baseline (speedup 1.0000x reference)
import functools

import jax
import jax.numpy as jnp
from jax import lax
from jax.experimental import pallas as pl
from jax.experimental.pallas import tpu as pltpu
from jax.experimental.pallas import tpu_sc as plsc

N_EDGES = 160000
N_NODES = 10000
D = 256
LN2 = 0.6931471805599453

N_TILES = 32
NPT = 320
ACC = NPT + 8
N_OUT_PAD = NPT * N_TILES
CH = 128
SEG_ROWS = N_EDGES // 16

BE = 2000


def _ssp(x):
    return jnp.logaddexp(x, 0.0) - LN2


def _mlp_body(seg_ref, x_ref, w1_ref, b1_ref, w2_ref, b2_ref,
              out_ref, splits_ref):
    x = x_ref[...]
    h = _ssp(jnp.dot(x, w1_ref[...], preferred_element_type=jnp.float32)
             + b1_ref[...])
    w = _ssp(jnp.dot(h, w2_ref[...], preferred_element_type=jnp.float32)
             + b2_ref[...])
    out_ref[...] = w

    @pl.when(pl.program_id(0) == 0)
    def _():
        seg = seg_ref[...]
        for k in range(48):
            cnt = jnp.sum((seg < jnp.int32(NPT * k)).astype(jnp.int32))
            splits_ref[k] = cnt


def _mlp_call(seg2d, dijk, W1, b1, W2, b2):
    grid = (N_EDGES // BE,)
    return pl.pallas_call(
        _mlp_body,
        grid=grid,
        in_specs=[
            pl.BlockSpec((SEG_ROWS // 8, 128), lambda i: (0, 0)),
            pl.BlockSpec((BE, D), lambda i: (i, 0)),
            pl.BlockSpec((D, D), lambda i: (0, 0)),
            pl.BlockSpec((1, D), lambda i: (0, 0)),
            pl.BlockSpec((D, D), lambda i: (0, 0)),
            pl.BlockSpec((1, D), lambda i: (0, 0)),
        ],
        out_specs=[
            pl.BlockSpec((BE, D), lambda i: (i, 0)),
            pl.BlockSpec(memory_space=pltpu.SMEM),
        ],
        out_shape=[
            jax.ShapeDtypeStruct((N_EDGES, D), jnp.float32),
            jax.ShapeDtypeStruct((48,), jnp.int32),
        ],
    )(seg2d, dijk, W1, b1, W2, b2)


def _sc_segsum_body(w_hbm, seg_hbm, splits_hbm, zeros_hbm, out_hbm,
                    acc, buf, segb, idxb, spl):
    c = lax.axis_index("c")
    s = lax.axis_index("s")
    wid = s * 2 + c
    base = NPT * wid

    pltpu.sync_copy(zeros_hbm, acc)

    pltpu.sync_copy(splits_hbm, spl)
    iot = lax.iota(jnp.int32, 16)
    sv2 = spl[pl.ds(wid, 16)]
    start = sv2[0]
    end = sv2[1]
    r0 = (start // 128) * 8
    r1 = (end + 15) // 16
    nch = jnp.maximum(0, (r1 - r0 + 7) // 8)

    lane15 = iot == 15

    def chunk(k, wm):
        rs = jnp.minimum(r0 + 8 * k, SEG_ROWS - 8)
        pltpu.sync_copy(w_hbm.at[pl.ds(rs * 16 * D, CH * D)], buf)
        pltpu.sync_copy(seg_hbm.at[pl.ds(rs * 16, 144)], segb)
        for j in range(8):
            sv = segb[pl.ds(16 * j, 16)]
            svn = segb[pl.ds(16 * j + 1, 16)]
            eid = (rs + j) * 16 + iot
            li_raw = sv - base
            ok = (li_raw >= 0) & (li_raw < NPT) & (eid >= wm)
            li = jnp.where(ok, li_raw, NPT)
            lin_raw = svn - base
            okn = (lin_raw >= 0) & (lin_raw < NPT) & (eid + 1 >= wm)
            lin = jnp.where(okn, lin_raw, NPT)
            idxb[pl.ds(16 * j, 16)] = li * D

        def edge(k, _):
            e = k
            base_off = idxb[pl.ds(e, 16)][0]
            for q in range(D // 16):
                v = buf[pl.ds(e * D + 16 * q, 16)]
                plsc.addupdate(acc.at[pl.ds(base_off + 16 * q, 16)], v)
            return 0

        lax.fori_loop(0, CH, edge, 0)
        return (rs + 8) * 16

    lax.fori_loop(0, nch, chunk, jnp.int32(0))

    pltpu.sync_copy(acc.at[pl.ds(0, NPT * D)],
                    out_hbm.at[pl.ds(base * D, NPT * D)])


def _sc_segsum(w_ijk, seg16, splits, zeros):
    mesh = plsc.VectorSubcoreMesh(core_axis_name="c", subcore_axis_name="s")
    f = functools.partial(
        pl.kernel,
        mesh=mesh,
        compiler_params=pltpu.CompilerParams(needs_layout_passes=False),
        out_type=jax.ShapeDtypeStruct((N_OUT_PAD * D,), jnp.float32),
        scratch_types=[
            pltpu.VMEM((ACC * D,), jnp.float32),
            pltpu.VMEM((CH * D,), jnp.float32),
            pltpu.VMEM((144,), jnp.int32),
            pltpu.VMEM((CH + 16,), jnp.int32),
            pltpu.VMEM((48,), jnp.int32),
        ],
    )(_sc_segsum_body)
    return f(w_ijk.reshape(-1), seg16.reshape(-1), splits, zeros.reshape(-1))


def kernel(dijk, seg_j, W1, b1, W2, b2):
    seg_j = seg_j.astype(jnp.int32)
    seg2d = seg_j.reshape(SEG_ROWS // 8, 128)
    seg16 = jnp.concatenate([seg_j, jnp.full((144,), 2**30, jnp.int32)])
    w_ijk, splits = _mlp_call(seg2d, dijk, W1, b1.reshape(1, D),
                              W2, b2.reshape(1, D))
    zeros = jnp.zeros((ACC, D), jnp.float32)
    w_pad = _sc_segsum(w_ijk, seg16, splits, zeros).reshape(N_OUT_PAD, D)
    return w_pad[:N_NODES]


# speedup vs baseline: 1.4112x; 1.4112x over previous
import functools

import jax
import jax.numpy as jnp
from jax import lax
from jax.experimental import pallas as pl
from jax.experimental.pallas import tpu as pltpu
from jax.experimental.pallas import tpu_sc as plsc

N_EDGES = 160000
N_NODES = 10000
D = 256
LN2 = 0.6931471805599453

N_TILES = 32
NPT = 320
ACC = NPT + 8
N_OUT_PAD = NPT * N_TILES
CH = 128
SEG_ROWS = N_EDGES // 16

BE = 2000


def _ssp(x):
    return jnp.logaddexp(x, 0.0) - LN2


def _mlp_body(seg_ref, x_ref, w1_ref, b1_ref, w2_ref, b2_ref,
              out_ref, splits_ref):
    x = x_ref[...]
    h = _ssp(jnp.dot(x, w1_ref[...], preferred_element_type=jnp.float32)
             + b1_ref[...])
    w = _ssp(jnp.dot(h, w2_ref[...], preferred_element_type=jnp.float32)
             + b2_ref[...])
    out_ref[...] = w

    @pl.when(pl.program_id(0) == 0)
    def _():
        seg = seg_ref[...]
        for k in range(48):
            cnt = jnp.sum((seg < jnp.int32(NPT * k)).astype(jnp.int32))
            splits_ref[k] = cnt


def _mlp_call(seg2d, dijk, W1, b1, W2, b2):
    grid = (N_EDGES // BE,)
    return pl.pallas_call(
        _mlp_body,
        grid=grid,
        in_specs=[
            pl.BlockSpec((SEG_ROWS // 8, 128), lambda i: (0, 0)),
            pl.BlockSpec((BE, D), lambda i: (i, 0)),
            pl.BlockSpec((D, D), lambda i: (0, 0)),
            pl.BlockSpec((1, D), lambda i: (0, 0)),
            pl.BlockSpec((D, D), lambda i: (0, 0)),
            pl.BlockSpec((1, D), lambda i: (0, 0)),
        ],
        out_specs=[
            pl.BlockSpec((BE, D), lambda i: (i, 0)),
            pl.BlockSpec(memory_space=pltpu.SMEM),
        ],
        out_shape=[
            jax.ShapeDtypeStruct((N_EDGES, D), jnp.float32),
            jax.ShapeDtypeStruct((48,), jnp.int32),
        ],
    )(seg2d, dijk, W1, b1, W2, b2)


def _sc_segsum_body(w_hbm, seg_hbm, splits_hbm, zeros_hbm, out_hbm,
                    acc, buf, segb, idxb, spl):
    c = lax.axis_index("c")
    s = lax.axis_index("s")
    wid = s * 2 + c
    base = NPT * wid

    pltpu.sync_copy(zeros_hbm, acc)

    pltpu.sync_copy(splits_hbm, spl)
    iot = lax.iota(jnp.int32, 16)
    sv2 = spl[pl.ds(wid, 16)]
    start = sv2[0]
    end = sv2[1]
    r0 = (start // 128) * 8
    r1 = (end + 15) // 16
    nch = jnp.maximum(0, (r1 - r0 + 7) // 8)

    lane15 = iot == 15

    def chunk(k, wm):
        rs = jnp.minimum(r0 + 8 * k, SEG_ROWS - 8)
        pltpu.sync_copy(w_hbm.at[pl.ds(rs * 16 * D, CH * D)], buf)
        pltpu.sync_copy(seg_hbm.at[pl.ds(rs * 16, 144)], segb)
        for j in range(8):
            sv = segb[pl.ds(16 * j, 16)]
            svn = segb[pl.ds(16 * j + 1, 16)]
            eid = (rs + j) * 16 + iot
            li_raw = sv - base
            ok = (li_raw >= 0) & (li_raw < NPT) & (eid >= wm)
            li = jnp.where(ok, li_raw, NPT)
            lin_raw = svn - base
            okn = (lin_raw >= 0) & (lin_raw < NPT) & (eid + 1 >= wm)
            lin = jnp.where(okn, lin_raw, NPT)
            idxb[pl.ds(16 * j, 16)] = li * D

        @plsc.parallel_loop(0, CH, 1, unroll=4)
        def _edge(e):
            base_off = idxb[pl.ds(e, 16)][0]
            for q in range(D // 16):
                v = buf[pl.ds(e * D + 16 * q, 16)]
                plsc.addupdate(acc.at[pl.ds(base_off + 16 * q, 16)], v)

        return (rs + 8) * 16

    lax.fori_loop(0, nch, chunk, jnp.int32(0))

    pltpu.sync_copy(acc.at[pl.ds(0, NPT * D)],
                    out_hbm.at[pl.ds(base * D, NPT * D)])


def _sc_segsum(w_ijk, seg16, splits, zeros):
    mesh = plsc.VectorSubcoreMesh(core_axis_name="c", subcore_axis_name="s")
    f = functools.partial(
        pl.kernel,
        mesh=mesh,
        compiler_params=pltpu.CompilerParams(needs_layout_passes=False),
        out_type=jax.ShapeDtypeStruct((N_OUT_PAD * D,), jnp.float32),
        scratch_types=[
            pltpu.VMEM((ACC * D,), jnp.float32),
            pltpu.VMEM((CH * D,), jnp.float32),
            pltpu.VMEM((144,), jnp.int32),
            pltpu.VMEM((CH + 16,), jnp.int32),
            pltpu.VMEM((48,), jnp.int32),
        ],
    )(_sc_segsum_body)
    return f(w_ijk.reshape(-1), seg16.reshape(-1), splits, zeros.reshape(-1))


def kernel(dijk, seg_j, W1, b1, W2, b2):
    seg_j = seg_j.astype(jnp.int32)
    seg2d = seg_j.reshape(SEG_ROWS // 8, 128)
    seg16 = jnp.concatenate([seg_j, jnp.full((144,), 2**30, jnp.int32)])
    w_ijk, splits = _mlp_call(seg2d, dijk, W1, b1.reshape(1, D),
                              W2, b2.reshape(1, D))
    zeros = jnp.zeros((ACC, D), jnp.float32)
    w_pad = _sc_segsum(w_ijk, seg16, splits, zeros).reshape(N_OUT_PAD, D)
    return w_pad[:N_NODES]
